# baseline (speedup 1.0000x reference)
import functools

import jax
import jax.numpy as jnp
from jax import lax
from jax.experimental import pallas as pl
from jax.experimental.pallas import tpu as pltpu
from jax.experimental.pallas import tpu_sc as plsc

NUM_SEQ = 16384
SEQ = 50
DIM = 64
NC = 2
NS = 16
NW = NC * NS
CHUNK = 128
NBLK = NUM_SEQ * SEQ // CHUNK
BLOCKS_PER_W = NBLK // NW
G = 5
NGRP = BLOCKS_PER_W // G
NBUF = 2

_mesh = plsc.VectorSubcoreMesh(core_axis_name="c", subcore_axis_name="s")


@functools.partial(
    pl.kernel,
    mesh=_mesh,
    out_type=jax.ShapeDtypeStruct((NBLK, CHUNK, DIM), jnp.float32),
    scratch_types=[
        pltpu.VMEM((BLOCKS_PER_W, CHUNK), jnp.int32),
        pltpu.VMEM((NBUF, G, CHUNK, DIM), jnp.float32),
        pltpu.SemaphoreType.DMA((NBUF,)),
        pltpu.SemaphoreType.DMA((NBUF,)),
    ],
    compiler_params=pltpu.CompilerParams(use_tc_tiling_on_sc=False),
)
def _embed_kernel(ids_hbm, table_hbm, out_hbm, idx_v, rows_v, sem_g, sem_w):
    wid = lax.axis_index("s") * NC + lax.axis_index("c")
    r0 = wid * BLOCKS_PER_W
    pltpu.sync_copy(ids_hbm.at[pl.ds(r0, BLOCKS_PER_W)], idx_v)

    def gather_descs(buf, grp):
        return [
            pltpu.make_async_copy(
                table_hbm.at[idx_v.at[grp * G + j]],
                rows_v.at[buf, j],
                sem_g.at[buf],
            )
            for j in range(G)
        ]

    def write_desc(buf, grp):
        return pltpu.make_async_copy(
            rows_v.at[buf], out_hbm.at[pl.ds(r0 + grp * G, G)], sem_w.at[buf]
        )

    def start_gathers(buf, grp):
        for d in gather_descs(buf, grp):
            d.start()

    def step(buf, grp):
        for d in gather_descs(buf, grp):
            d.wait()
        write_desc(buf, grp).start()
        write_desc(buf, grp).wait()

        @pl.when(grp + NBUF < NGRP)
        def _():
            start_gathers(buf, grp + NBUF)

    for b in range(NBUF):
        start_gathers(b, b)

    def body(t, carry):
        for b in range(NBUF):
            step(b, t * NBUF + b)
        return carry

    lax.fori_loop(0, NGRP // NBUF, body, 0)


def kernel(input_ids, weight):
    ids = input_ids.astype(jnp.int32).reshape(NBLK, CHUNK)
    out = _embed_kernel(ids, weight)
    return out.reshape(NUM_SEQ, SEQ, DIM)
